# baseline (speedup 1.0000x reference)
import functools

import jax
import jax.numpy as jnp
from jax import lax
from jax.experimental import pallas as pl
from jax.experimental.pallas import tpu as pltpu
from jax.experimental.pallas import tpu_sc as plsc

B = 1024
S = 256
H = 14
W = 14
C = 128
D = H * W * C
K = 49
DK = D // K
NC = 2
NS = 16
NW = NC * NS
BPW = B // NW
LANES = 16


def kernel(inputs, injection_sites, masks, indices):
    tbl_inj = injection_sites.reshape(S * K, DK)
    tbl_msk = masks.reshape(S * K, DK)
    x3 = inputs.reshape(B, K, DK)

    mesh = plsc.VectorSubcoreMesh(core_axis_name="c", subcore_axis_name="s")

    @functools.partial(
        pl.kernel,
        mesh=mesh,
        out_type=jax.ShapeDtypeStruct((B, K, DK), jnp.float32),
        scratch_types=[
            pltpu.VMEM((BPW,), jnp.int32),
            pltpu.VMEM((BPW,), jnp.int32),
            pltpu.VMEM((BPW, DK), jnp.float32),
            pltpu.VMEM((BPW, DK), jnp.float32),
            pltpu.VMEM((BPW, DK), jnp.float32),
            pltpu.SemaphoreType.DMA,
            pltpu.SemaphoreType.DMA,
            pltpu.SemaphoreType.DMA,
        ],
    )
    def run(inj_hbm, msk_hbm, x_hbm, idx_hbm, out_hbm,
            idxraw_v, idxk_v, x_v, m_v, a_v, sem_a, sem_m, sem_x):
        wid = lax.axis_index("s") * NC + lax.axis_index("c")
        base = wid * BPW
        pltpu.sync_copy(idx_hbm.at[pl.ds(base, BPW)], idxraw_v)

        @pl.loop(0, K)
        def _(kk):
            @pl.loop(0, BPW, step=LANES)
            def _(j):
                idxk_v[pl.ds(j, LANES)] = idxraw_v[pl.ds(j, LANES)] * K + kk

            cp_a = pltpu.async_copy(inj_hbm.at[idxk_v], a_v, sem_a)
            cp_m = pltpu.async_copy(msk_hbm.at[idxk_v], m_v, sem_m)
            cp_x = pltpu.async_copy(x_hbm.at[pl.ds(base, BPW), kk], x_v, sem_x)
            cp_a.wait()
            cp_m.wait()
            cp_x.wait()

            @pl.loop(0, BPW)
            def _(i):
                @pl.loop(0, DK, step=LANES)
                def _(j):
                    sl = (i, pl.ds(j, LANES))
                    x_v.at[*sl][...] = (x_v.at[*sl][...] * m_v.at[*sl][...]
                                        + a_v.at[*sl][...])

            pltpu.sync_copy(x_v, out_hbm.at[pl.ds(base, BPW), kk])

    out = run(tbl_inj, tbl_msk, x3, indices)
    return out.reshape(B, H, W, C)


# speedup vs baseline: 2.7452x; 2.7452x over previous
import jax
import jax.numpy as jnp
from jax import lax
from jax.experimental import pallas as pl
from jax.experimental.pallas import tpu as pltpu
from jax.experimental.pallas import tpu_sc as plsc

B = 1024
S = 256
H = 14
W = 14
C = 128
NC = 2
NS = 16
NW = NC * NS
BPW = B // NW
G = 8
PG = LANES = 16
NGRP = PG // G
NSTEP = (BPW // G) * H
CV = C // LANES


def kernel(inputs, injection_sites, masks, indices):
    tbl_inj = injection_sites.reshape(S * H, W, C)
    tbl_msk = masks.reshape(S * H, W, C)

    mesh = plsc.VectorSubcoreMesh(core_axis_name="c", subcore_axis_name="s")

    def run(inj_hbm, msk_hbm, x_hbm, idx_hbm, out_hbm,
            idxraw_v, exp_v, x_v, m_v, a_v, o_v, sems, sem_wb):
        wid = lax.axis_index("s") * NC + lax.axis_index("c")
        base = wid * BPW
        pltpu.sync_copy(idx_hbm.at[pl.ds(base, BPW)], idxraw_v)

        for t in range(BPW // PG):
            idx16 = idxraw_v[pl.ds(t * PG, PG)] * H
            for h in range(H):
                exp_v[pl.ds((t * H + h) * PG, PG)] = idx16 + h

        def coords(n):
            t = n // (NGRP * H)
            r = lax.rem(n, NGRP * H)
            h = r // NGRP
            p = lax.rem(r, NGRP)
            return t, h, p

        def issue(n, j):
            t, h, p = coords(n)
            idx8 = exp_v.at[pl.ds((t * H + h) * PG + p * G, G)]
            b0 = base + t * PG + p * G
            pltpu.async_copy(inj_hbm.at[idx8], a_v.at[j], sems.at[j, 0])
            pltpu.async_copy(msk_hbm.at[idx8], m_v.at[j], sems.at[j, 1])
            pltpu.async_copy(x_hbm.at[pl.ds(b0, G), h], x_v.at[j], sems.at[j, 2])

        def wait_in(j):
            pltpu.make_async_copy(inj_hbm.at[pl.ds(0, G)], a_v.at[j],
                                  sems.at[j, 0]).wait()
            pltpu.make_async_copy(msk_hbm.at[pl.ds(0, G)], m_v.at[j],
                                  sems.at[j, 1]).wait()
            pltpu.make_async_copy(x_hbm.at[pl.ds(0, G), 0], x_v.at[j],
                                  sems.at[j, 2]).wait()

        def compute(j):
            xp, mp, ap = x_v.at[j], m_v.at[j], a_v.at[j]

            @pl.loop(0, G * W)
            def _(q):
                i = q // W
                w = lax.rem(q, W)
                for c in range(CV):
                    sl = (i, w, pl.ds(c * LANES, LANES))
                    o_v.at[*sl][...] = (xp.at[*sl][...] * mp.at[*sl][...]
                                        + ap.at[*sl][...])

        def wb_start(n):
            t, h, p = coords(n)
            b0 = base + t * PG + p * G
            pltpu.async_copy(o_v, out_hbm.at[pl.ds(b0, G), h], sem_wb)

        def wb_wait():
            pltpu.make_async_copy(o_v, out_hbm.at[pl.ds(0, G), 0],
                                  sem_wb).wait()

        def step(n, j, wait_prev_wb, issue_next):
            wait_in(j)
            if wait_prev_wb:
                wb_wait()
            compute(j)
            wb_start(n)
            if issue_next:
                issue(n + 2, j)

        issue(0, 0)
        issue(1, 1)
        step(0, 0, False, True)
        step(1, 1, True, True)

        @pl.loop(2, NSTEP - 2, step=2)
        def _(n0):
            step(n0, 0, True, True)
            step(n0 + 1, 1, True, True)

        step(NSTEP - 2, 0, True, False)
        step(NSTEP - 1, 1, True, False)
        wb_wait()

    grid_kernel = pl.kernel(
        run,
        out_type=jax.ShapeDtypeStruct((B, H, W, C), jnp.float32),
        mesh=mesh,
        scratch_types=[
            pltpu.VMEM((BPW,), jnp.int32),
            pltpu.VMEM((BPW * H,), jnp.int32),
            pltpu.VMEM((2, G, W, C), jnp.float32),
            pltpu.VMEM((2, G, W, C), jnp.float32),
            pltpu.VMEM((2, G, W, C), jnp.float32),
            pltpu.VMEM((G, W, C), jnp.float32),
            pltpu.SemaphoreType.DMA((2, 3)),
            pltpu.SemaphoreType.DMA,
        ],
        compiler_params=pltpu.CompilerParams(use_tc_tiling_on_sc=True),
    )
    return grid_kernel(tbl_inj, tbl_msk, inputs, indices)


# speedup vs baseline: 5.4210x; 1.9747x over previous
import jax
import jax.numpy as jnp
from jax import lax
from jax.experimental import pallas as pl
from jax.experimental.pallas import tpu as pltpu
from jax.experimental.pallas import tpu_sc as plsc

B = 1024
S = 256
H = 14
W = 14
C = 128
Q = H * W
NR = Q * B
NC = 2
NS = 16
NW = NC * NS
RPW = NR // NW
RCH = 64
NSTEP = RPW // RCH
LANES = 16
CV = C // LANES


def kernel(inputs, injection_sites, masks, indices):
    inj2 = jnp.transpose(injection_sites, (1, 2, 0, 3)).reshape(Q * S, C)
    msk2 = jnp.transpose(masks, (1, 2, 0, 3)).reshape(Q * S, C)
    x2 = jnp.transpose(inputs, (1, 2, 0, 3)).reshape(NR, C)

    mesh = plsc.VectorSubcoreMesh(core_axis_name="c", subcore_axis_name="s")

    def run(inj_hbm, msk_hbm, x_hbm, idx_hbm, out_hbm,
            idx_v, exp_v, x_v, m_v, a_v, o_v, sems, sem_wb):
        wid = lax.axis_index("s") * NC + lax.axis_index("c")
        base = wid * RPW
        pltpu.sync_copy(idx_hbm, idx_v)

        lane = lax.iota(jnp.int32, LANES)

        @pl.loop(0, RPW, step=LANES)
        def _(t):
            r16 = base + t + lane
            q = lax.shift_right_logical(r16, 10)
            b = lax.bitwise_and(r16, 1023)
            v = plsc.load_gather(idx_v, [b])
            exp_v[pl.ds(t, LANES)] = lax.shift_left(q, 8) + v

        def issue(n, j):
            idxs = exp_v.at[pl.ds(n * RCH, RCH)]
            r0 = base + n * RCH
            pltpu.async_copy(inj_hbm.at[idxs], a_v.at[j], sems.at[j, 0])
            pltpu.async_copy(msk_hbm.at[idxs], m_v.at[j], sems.at[j, 1])
            pltpu.async_copy(x_hbm.at[pl.ds(r0, RCH)], x_v.at[j], sems.at[j, 2])

        def wait_in(j):
            pltpu.make_async_copy(inj_hbm.at[pl.ds(0, RCH)], a_v.at[j],
                                  sems.at[j, 0]).wait()
            pltpu.make_async_copy(msk_hbm.at[pl.ds(0, RCH)], m_v.at[j],
                                  sems.at[j, 1]).wait()
            pltpu.make_async_copy(x_hbm.at[pl.ds(0, RCH)], x_v.at[j],
                                  sems.at[j, 2]).wait()

        def compute(j):
            xp, mp, ap = x_v.at[j], m_v.at[j], a_v.at[j]

            @pl.loop(0, RCH)
            def _(i):
                for c in range(CV):
                    sl = (i, pl.ds(c * LANES, LANES))
                    o_v.at[*sl][...] = (xp.at[*sl][...] * mp.at[*sl][...]
                                        + ap.at[*sl][...])

        def wb_start(n):
            r0 = base + n * RCH
            pltpu.async_copy(o_v, out_hbm.at[pl.ds(r0, RCH)], sem_wb)

        def wb_wait():
            pltpu.make_async_copy(o_v, out_hbm.at[pl.ds(0, RCH)],
                                  sem_wb).wait()

        def step(n, j, wait_prev_wb, issue_next):
            wait_in(j)
            if wait_prev_wb:
                wb_wait()
            compute(j)
            wb_start(n)
            if issue_next:
                issue(n + 2, j)

        issue(0, 0)
        issue(1, 1)
        step(0, 0, False, True)
        step(1, 1, True, True)

        @pl.loop(2, NSTEP - 2, step=2)
        def _(n0):
            step(n0, 0, True, True)
            step(n0 + 1, 1, True, True)

        step(NSTEP - 2, 0, True, False)
        step(NSTEP - 1, 1, True, False)
        wb_wait()

    grid_kernel = pl.kernel(
        run,
        out_type=jax.ShapeDtypeStruct((NR, C), jnp.float32),
        mesh=mesh,
        scratch_types=[
            pltpu.VMEM((B,), jnp.int32),
            pltpu.VMEM((RPW,), jnp.int32),
            pltpu.VMEM((2, RCH, C), jnp.float32),
            pltpu.VMEM((2, RCH, C), jnp.float32),
            pltpu.VMEM((2, RCH, C), jnp.float32),
            pltpu.VMEM((RCH, C), jnp.float32),
            pltpu.SemaphoreType.DMA((2, 3)),
            pltpu.SemaphoreType.DMA,
        ],
        compiler_params=pltpu.CompilerParams(use_tc_tiling_on_sc=True,
                                             needs_layout_passes=False),
    )
    out2 = grid_kernel(inj2, msk2, x2, indices)
    return jnp.transpose(out2.reshape(H, W, B, C), (2, 0, 1, 3))
